# baseline (speedup 1.0000x reference)
import functools

import jax
import jax.numpy as jnp
from jax import lax
from jax.experimental import pallas as pl
from jax.experimental.pallas import tpu as pltpu
from jax.experimental.pallas import tpu_sc as plsc

N = 10000
E = 320000
D = 128
LANES = 16
NSUB = 16
NW = 2 * NSUB
EDGES_PER_TILE = E // NSUB
CHUNK = 80
NCHUNKS = EDGES_PER_TILE // CHUNK
NCHUNKS_P = 256
NSEG = 4
SEGCH = NCHUNKS_P // NSEG
SEGPAIR = SEGCH // 2
ROWS_PER_TILE = 624
FVECS = D // LANES
GROUPS = CHUNK // LANES


def _mm_body(x_ref, w1_ref, w2_ref, o_ref):
    x = x_ref[...]
    o_ref[0] = jnp.dot(x, w1_ref[...], preferred_element_type=jnp.float32)
    o_ref[1] = jnp.dot(x, w2_ref[...], preferred_element_type=jnp.float32)


def _matmuls(x, W1, W2):
    BM = 1000
    return pl.pallas_call(
        _mm_body,
        grid=(N // BM,),
        in_specs=[
            pl.BlockSpec((BM, D), lambda i: (i, 0)),
            pl.BlockSpec((D, D), lambda i: (0, 0)),
            pl.BlockSpec((D, D), lambda i: (0, 0)),
        ],
        out_specs=pl.BlockSpec((2, BM, D), lambda i: (0, i, 0)),
        out_shape=jax.ShapeDtypeStruct((2, N, D), jnp.float32),
    )(x, W1, W2)


def _bcast_lane(vec, e2):
    return lax.gather(
        vec,
        jnp.full((LANES, 1), e2, jnp.int32),
        lax.GatherDimensionNumbers(
            offset_dims=(), collapsed_slice_dims=(0,), start_index_map=(0,)),
        (1,),
        mode=lax.GatherScatterMode.PROMISE_IN_BOUNDS)


def _sc_body(x12_ref, rslab_ref, cslab_ref, vslab_ref, out_ref,
             rslab_v, cslab_v, vslab_v, rows_a, rows_b, acc, sem_a, sem_b):
    c = lax.axis_index("c")
    s = lax.axis_index("s")
    wid = c * NSUB + s

    zero = jnp.zeros((LANES,), jnp.float32)

    def zero_body(i, carry):
        r = i // FVECS
        f = i % FVECS
        rows_a[r, pl.ds(f * LANES, LANES)] = zero
        return carry

    lax.fori_loop(0, CHUNK * FVECS, zero_body, 0)
    rbase = s * ROWS_PER_TILE
    for j in range(7):
        pltpu.sync_copy(rows_a, acc.at[pl.ds(rbase + j * CHUNK, CHUNK)])
    pltpu.sync_copy(rows_a.at[pl.ds(0, 64)],
                    acc.at[pl.ds(rbase + 7 * CHUNK, 64)])

    @pl.when(s == NSUB - 1)
    def _zero_tail():
        pltpu.sync_copy(rows_a.at[pl.ds(0, 16)],
                        acc.at[pl.ds(rbase + ROWS_PER_TILE, 16)])

    plsc.subcore_barrier()

    def scale(rows_v, i):
        def scale_body(g, carry2):
            valvec = vslab_v[i, pl.ds(g * LANES, LANES)]
            for e2 in range(LANES):
                bc = _bcast_lane(valvec, e2)
                row = g * LANES + e2
                for f in range(FVECS):
                    sl = pl.ds(f * LANES, LANES)
                    rows_v[row, sl] = rows_v[row, sl] * bc
            return carry2

        lax.fori_loop(0, GROUPS, scale_body, 0)

    def gather(rows_v, i, sem):
        pltpu.async_copy(x12_ref.at[cslab_v.at[i]], rows_v, sem)

    def gwait(rows_v, sem):
        pltpu.make_async_copy(x12_ref.at[pl.ds(0, CHUNK)], rows_v, sem).wait()

    def scatter(rows_v, i):
        pltpu.sync_copy(rows_v, acc.at[rslab_v.at[i]], add=True)

    def pair_body(j, carry):
        i0 = 2 * j
        gather(rows_b, i0 + 1, sem_b)
        gwait(rows_a, sem_a)
        scale(rows_a, i0)
        scatter(rows_a, i0)

        @pl.when(j < SEGPAIR - 1)
        def _prefetch_a():
            gather(rows_a, i0 + 2, sem_a)

        gwait(rows_b, sem_b)
        scale(rows_b, i0 + 1)
        scatter(rows_b, i0 + 1)
        return carry

    for seg in range(NSEG):
        sl = pl.ds(seg * SEGCH, SEGCH)
        pltpu.sync_copy(rslab_ref.at[wid, sl], rslab_v)
        pltpu.sync_copy(cslab_ref.at[wid, sl], cslab_v)
        pltpu.sync_copy(vslab_ref.at[wid, sl], vslab_v)
        gather(rows_a, 0, sem_a)
        lax.fori_loop(0, SEGPAIR, pair_body, 0)

    plsc.subcore_barrier()

    pltpu.sync_copy(acc.at[pl.ds(rbase, ROWS_PER_TILE)],
                    out_ref.at[c, pl.ds(rbase, ROWS_PER_TILE)])

    @pl.when(s == NSUB - 1)
    def _dump_tail():
        pltpu.sync_copy(acc.at[pl.ds(rbase + ROWS_PER_TILE, 16)],
                        out_ref.at[c, pl.ds(rbase + ROWS_PER_TILE, 16)])


def _sc_spmm(x12, rslab, cslab, vslab):
    mesh = plsc.VectorSubcoreMesh(core_axis_name="c", subcore_axis_name="s")
    f = pl.kernel(
        _sc_body,
        out_type=jax.ShapeDtypeStruct((2, N, D), jnp.float32),
        mesh=mesh,
        scratch_types=[
            pltpu.VMEM((SEGCH, CHUNK), jnp.int32),
            pltpu.VMEM((SEGCH, CHUNK), jnp.int32),
            pltpu.VMEM((SEGCH, CHUNK), jnp.float32),
            pltpu.VMEM((CHUNK, D), jnp.float32),
            pltpu.VMEM((CHUNK, D), jnp.float32),
            pltpu.VMEM_SHARED((N, D), jnp.float32),
            pltpu.SemaphoreType.DMA,
            pltpu.SemaphoreType.DMA,
        ],
    )
    return f(x12, rslab, cslab, vslab)


def _combine_body(a_ref, b_ref, o_ref):
    o_ref[...] = jnp.maximum(a_ref[...] + b_ref[...], 0.0)


def _relu_combine(a, b):
    BM = 1000
    return pl.pallas_call(
        _combine_body,
        grid=(N // BM,),
        in_specs=[
            pl.BlockSpec((BM, D), lambda i: (i, 0)),
            pl.BlockSpec((BM, D), lambda i: (i, 0)),
        ],
        out_specs=pl.BlockSpec((BM, D), lambda i: (i, 0)),
        out_shape=jax.ShapeDtypeStruct((N, D), jnp.float32),
    )(a, b)


def kernel(inputs, adj1_index, adj1_values, adj2_index, adj2_values, W1, W2):
    xw = _matmuls(inputs, W1, W2)
    x12 = xw.reshape(2 * N, D)
    rows = jnp.concatenate([adj1_index[0], adj2_index[0]])
    cols = jnp.concatenate([adj1_index[1], adj2_index[1] + N])
    vals = jnp.concatenate([adj1_values, adj2_values])
    pad = ((0, 0), (0, NCHUNKS_P - NCHUNKS), (0, 0))
    rslab = jnp.pad(rows.reshape(NW, NCHUNKS, CHUNK), pad)
    cslab = jnp.pad(cols.reshape(NW, NCHUNKS, CHUNK), pad)
    vslab = jnp.pad(vals.reshape(NW, NCHUNKS, CHUNK), pad)
    parts = _sc_spmm(x12, rslab, cslab, vslab)
    return _relu_combine(parts[0], parts[1])


# speedup vs baseline: 1.0362x; 1.0362x over previous
import functools

import jax
import jax.numpy as jnp
from jax import lax
from jax.experimental import pallas as pl
from jax.experimental.pallas import tpu as pltpu
from jax.experimental.pallas import tpu_sc as plsc

N = 10000
E = 320000
D = 128
LANES = 16
NSUB = 16
NW = 2 * NSUB
EDGES_PER_TILE = E // NSUB
CHUNK = 128
NCHUNKS = EDGES_PER_TILE // CHUNK
NCHUNKS_P = 160
NSEG = 4
SEGCH = NCHUNKS_P // NSEG
SEGPAIR = SEGCH // 2
ROWS_PER_TILE = 624
FVECS = D // LANES
GROUPS = CHUNK // LANES


def _mm_body(x_ref, w1_ref, w2_ref, o_ref):
    x = x_ref[...]
    o_ref[0] = jnp.dot(x, w1_ref[...], preferred_element_type=jnp.float32)
    o_ref[1] = jnp.dot(x, w2_ref[...], preferred_element_type=jnp.float32)


def _matmuls(x, W1, W2):
    BM = 1000
    return pl.pallas_call(
        _mm_body,
        grid=(N // BM,),
        in_specs=[
            pl.BlockSpec((BM, D), lambda i: (i, 0)),
            pl.BlockSpec((D, D), lambda i: (0, 0)),
            pl.BlockSpec((D, D), lambda i: (0, 0)),
        ],
        out_specs=pl.BlockSpec((2, BM, D), lambda i: (0, i, 0)),
        out_shape=jax.ShapeDtypeStruct((2, N, D), jnp.float32),
    )(x, W1, W2)


def _bcast_lane(vec, e2):
    return lax.gather(
        vec,
        jnp.full((LANES, 1), e2, jnp.int32),
        lax.GatherDimensionNumbers(
            offset_dims=(), collapsed_slice_dims=(0,), start_index_map=(0,)),
        (1,),
        mode=lax.GatherScatterMode.PROMISE_IN_BOUNDS)


def _sc_body(x12_ref, rslab_ref, cslab_ref, vslab_ref, out_ref,
             rslab_v, cslab_v, vslab_v, rows_a, rows_b, acc, sem_a, sem_b):
    c = lax.axis_index("c")
    s = lax.axis_index("s")
    wid = c * NSUB + s

    zero = jnp.zeros((LANES,), jnp.float32)

    def zero_body(i, carry):
        r = i // FVECS
        f = i % FVECS
        rows_a[r, pl.ds(f * LANES, LANES)] = zero
        return carry

    lax.fori_loop(0, CHUNK * FVECS, zero_body, 0)
    rbase = s * ROWS_PER_TILE
    for j in range(7):
        pltpu.sync_copy(rows_a, acc.at[pl.ds(rbase + j * CHUNK, CHUNK)])
    pltpu.sync_copy(rows_a.at[pl.ds(0, 64)],
                    acc.at[pl.ds(rbase + 7 * CHUNK, 64)])

    @pl.when(s == NSUB - 1)
    def _zero_tail():
        pltpu.sync_copy(rows_a.at[pl.ds(0, 16)],
                        acc.at[pl.ds(rbase + ROWS_PER_TILE, 16)])

    plsc.subcore_barrier()

    def scale(rows_v, i):
        def scale_body(g, carry2):
            valvec = vslab_v[i, pl.ds(g * LANES, LANES)]
            for e2 in range(LANES):
                bc = _bcast_lane(valvec, e2)
                row = g * LANES + e2
                for f in range(FVECS):
                    sl = pl.ds(f * LANES, LANES)
                    rows_v[row, sl] = rows_v[row, sl] * bc
            return carry2

        lax.fori_loop(0, GROUPS, scale_body, 0)

    def gather(rows_v, i, sem):
        pltpu.async_copy(x12_ref.at[cslab_v.at[i]], rows_v, sem)

    def gwait(rows_v, sem):
        pltpu.make_async_copy(x12_ref.at[pl.ds(0, CHUNK)], rows_v, sem).wait()

    def scatter(rows_v, i):
        pltpu.sync_copy(rows_v, acc.at[rslab_v.at[i]], add=True)

    def pair_body(j, carry):
        i0 = 2 * j
        gather(rows_b, i0 + 1, sem_b)
        gwait(rows_a, sem_a)
        scale(rows_a, i0)
        scatter(rows_a, i0)

        @pl.when(j < SEGPAIR - 1)
        def _prefetch_a():
            gather(rows_a, i0 + 2, sem_a)

        gwait(rows_b, sem_b)
        scale(rows_b, i0 + 1)
        scatter(rows_b, i0 + 1)
        return carry

    for seg in range(NSEG):
        sl = pl.ds(seg * SEGCH, SEGCH)
        pltpu.sync_copy(rslab_ref.at[wid, sl], rslab_v)
        pltpu.sync_copy(cslab_ref.at[wid, sl], cslab_v)
        pltpu.sync_copy(vslab_ref.at[wid, sl], vslab_v)
        gather(rows_a, 0, sem_a)
        lax.fori_loop(0, SEGPAIR, pair_body, 0)

    plsc.subcore_barrier()

    pltpu.sync_copy(acc.at[pl.ds(rbase, ROWS_PER_TILE)],
                    out_ref.at[c, pl.ds(rbase, ROWS_PER_TILE)])

    @pl.when(s == NSUB - 1)
    def _dump_tail():
        pltpu.sync_copy(acc.at[pl.ds(rbase + ROWS_PER_TILE, 16)],
                        out_ref.at[c, pl.ds(rbase + ROWS_PER_TILE, 16)])


def _sc_spmm(x12, rslab, cslab, vslab):
    mesh = plsc.VectorSubcoreMesh(core_axis_name="c", subcore_axis_name="s")
    f = pl.kernel(
        _sc_body,
        out_type=jax.ShapeDtypeStruct((2, N, D), jnp.float32),
        mesh=mesh,
        scratch_types=[
            pltpu.VMEM((SEGCH, CHUNK), jnp.int32),
            pltpu.VMEM((SEGCH, CHUNK), jnp.int32),
            pltpu.VMEM((SEGCH, CHUNK), jnp.float32),
            pltpu.VMEM((CHUNK, D), jnp.float32),
            pltpu.VMEM((CHUNK, D), jnp.float32),
            pltpu.VMEM_SHARED((N, D), jnp.float32),
            pltpu.SemaphoreType.DMA,
            pltpu.SemaphoreType.DMA,
        ],
    )
    return f(x12, rslab, cslab, vslab)


def _combine_body(a_ref, b_ref, o_ref):
    o_ref[...] = jnp.maximum(a_ref[...] + b_ref[...], 0.0)


def _relu_combine(a, b):
    BM = 1000
    return pl.pallas_call(
        _combine_body,
        grid=(N // BM,),
        in_specs=[
            pl.BlockSpec((BM, D), lambda i: (i, 0)),
            pl.BlockSpec((BM, D), lambda i: (i, 0)),
        ],
        out_specs=pl.BlockSpec((BM, D), lambda i: (i, 0)),
        out_shape=jax.ShapeDtypeStruct((N, D), jnp.float32),
    )(a, b)


def kernel(inputs, adj1_index, adj1_values, adj2_index, adj2_values, W1, W2):
    xw = _matmuls(inputs, W1, W2)
    x12 = xw.reshape(2 * N, D)
    rows = jnp.concatenate([adj1_index[0], adj2_index[0]])
    cols = jnp.concatenate([adj1_index[1], adj2_index[1] + N])
    vals = jnp.concatenate([adj1_values, adj2_values])
    pad = ((0, 0), (0, NCHUNKS_P * CHUNK - EDGES_PER_TILE))
    rslab = jnp.pad(rows.reshape(NW, EDGES_PER_TILE), pad).reshape(
        NW, NCHUNKS_P, CHUNK)
    cslab = jnp.pad(cols.reshape(NW, EDGES_PER_TILE), pad).reshape(
        NW, NCHUNKS_P, CHUNK)
    vslab = jnp.pad(vals.reshape(NW, EDGES_PER_TILE), pad).reshape(
        NW, NCHUNKS_P, CHUNK)
    parts = _sc_spmm(x12, rslab, cslab, vslab)
    return _relu_combine(parts[0], parts[1])


# speedup vs baseline: 1.0848x; 1.0470x over previous
import jax
import jax.numpy as jnp
from jax import lax
from jax.experimental import pallas as pl
from jax.experimental.pallas import tpu as pltpu
from jax.experimental.pallas import tpu_sc as plsc

N = 10000
E = 320000
D = 128
LANES = 16
NSUB = 16
EDGES_PER_TILE = E // NSUB
CHUNK = 80
NCHUNKS = EDGES_PER_TILE // CHUNK
ROWS_PER_TILE = 624
ZROWS = 80
FVECS = D // LANES


def _mm_body(x_ref, w1_ref, w2_ref, o_ref):
    x = x_ref[...]
    o_ref[0] = jnp.dot(x, w1_ref[...], preferred_element_type=jnp.float32)
    o_ref[1] = jnp.dot(x, w2_ref[...], preferred_element_type=jnp.float32)


def _matmuls(x, W1, W2):
    BM = 1000
    return pl.pallas_call(
        _mm_body,
        grid=(N // BM,),
        in_specs=[
            pl.BlockSpec((BM, D), lambda i: (i, 0)),
            pl.BlockSpec((D, D), lambda i: (0, 0)),
            pl.BlockSpec((D, D), lambda i: (0, 0)),
        ],
        out_specs=pl.BlockSpec((2, BM, D), lambda i: (0, i, 0)),
        out_shape=jax.ShapeDtypeStruct((2, N, D), jnp.float32),
    )(x, W1, W2)


def _sc_body(x12_ref, row_ref, col_ref, val_ref, out_ref,
             col_v, row_v, val_v, rows_v, zbuf_v, acc, sem):
    c = lax.axis_index("c")
    s = lax.axis_index("s")

    zero = jnp.zeros((LANES,), jnp.float32)

    def zero_body(i, carry):
        r = i // FVECS
        f = i % FVECS
        zbuf_v[r, pl.ds(f * LANES, LANES)] = zero
        return carry

    lax.fori_loop(0, ZROWS * FVECS, zero_body, 0)
    rbase = s * ROWS_PER_TILE
    for j in range(7):
        pltpu.sync_copy(zbuf_v, acc.at[pl.ds(rbase + j * ZROWS, ZROWS)])
    pltpu.sync_copy(zbuf_v.at[pl.ds(0, 64)],
                    acc.at[pl.ds(rbase + 7 * ZROWS, 64)])

    @pl.when(s == NSUB - 1)
    def _zero_tail():
        pltpu.sync_copy(zbuf_v.at[pl.ds(0, 16)],
                        acc.at[pl.ds(rbase + ROWS_PER_TILE, 16)])

    plsc.subcore_barrier()

    ebase = (c * NSUB + s) * EDGES_PER_TILE

    def chunk_body(i, carry):
        off = ebase + i * CHUNK
        pltpu.sync_copy(col_ref.at[pl.ds(off, CHUNK)], col_v)
        pltpu.sync_copy(row_ref.at[pl.ds(off, CHUNK)], row_v)
        pltpu.sync_copy(val_ref.at[pl.ds(off, CHUNK)], val_v)
        pltpu.async_copy(x12_ref.at[col_v], rows_v, sem).wait()

        def scale_body(g, carry2):
            valvec = val_v[pl.ds(g * LANES, LANES)]
            for e2 in range(LANES):
                bc = lax.gather(
                    valvec,
                    jnp.full((LANES, 1), e2, jnp.int32),
                    lax.GatherDimensionNumbers(
                        offset_dims=(), collapsed_slice_dims=(0,),
                        start_index_map=(0,)),
                    (1,),
                    mode=lax.GatherScatterMode.PROMISE_IN_BOUNDS)
                row = g * LANES + e2
                for f in range(FVECS):
                    sl = pl.ds(f * LANES, LANES)
                    rows_v[row, sl] = rows_v[row, sl] * bc
            return carry2

        lax.fori_loop(0, CHUNK // LANES, scale_body, 0)
        pltpu.sync_copy(rows_v, acc.at[row_v], add=True)
        return carry

    lax.fori_loop(0, NCHUNKS, chunk_body, 0)
    plsc.subcore_barrier()

    pltpu.sync_copy(acc.at[pl.ds(rbase, ROWS_PER_TILE)],
                    out_ref.at[c, pl.ds(rbase, ROWS_PER_TILE)])

    @pl.when(s == NSUB - 1)
    def _dump_tail():
        pltpu.sync_copy(acc.at[pl.ds(rbase + ROWS_PER_TILE, 16)],
                        out_ref.at[c, pl.ds(rbase + ROWS_PER_TILE, 16)])


def _sc_spmm(x12, rows, cols, vals):
    mesh = plsc.VectorSubcoreMesh(core_axis_name="c", subcore_axis_name="s")
    f = pl.kernel(
        _sc_body,
        out_type=jax.ShapeDtypeStruct((2, N, D), jnp.float32),
        mesh=mesh,
        scratch_types=[
            pltpu.VMEM((CHUNK,), jnp.int32),
            pltpu.VMEM((CHUNK,), jnp.int32),
            pltpu.VMEM((CHUNK,), jnp.float32),
            pltpu.VMEM((CHUNK, D), jnp.float32),
            pltpu.VMEM((ZROWS, D), jnp.float32),
            pltpu.VMEM_SHARED((N, D), jnp.float32),
            pltpu.SemaphoreType.DMA,
        ],
    )
    return f(x12, rows, cols, vals)


def _combine_body(a_ref, b_ref, o_ref):
    o_ref[...] = jnp.maximum(a_ref[...] + b_ref[...], 0.0)


def _relu_combine(a, b):
    BM = 1000
    return pl.pallas_call(
        _combine_body,
        grid=(N // BM,),
        in_specs=[
            pl.BlockSpec((BM, D), lambda i: (i, 0)),
            pl.BlockSpec((BM, D), lambda i: (i, 0)),
        ],
        out_specs=pl.BlockSpec((BM, D), lambda i: (i, 0)),
        out_shape=jax.ShapeDtypeStruct((N, D), jnp.float32),
    )(a, b)


def kernel(inputs, adj1_index, adj1_values, adj2_index, adj2_values, W1, W2):
    xw = _matmuls(inputs, W1, W2)
    x12 = xw.reshape(2 * N, D)
    rows = jnp.concatenate([adj1_index[0], adj2_index[0]])
    cols = jnp.concatenate([adj1_index[1], adj2_index[1] + N])
    vals = jnp.concatenate([adj1_values, adj2_values])
    parts = _sc_spmm(x12, rows, cols, vals)
    return _relu_combine(parts[0], parts[1])
